# Initial kernel scaffold; baseline (speedup 1.0000x reference)
import jax
import jax.numpy as jnp
from jax.experimental import pallas as pl


def kernel(x, src, dst, W_theta, W_phi, gamma, beta):
    raise NotImplementedError("write your pallas kernel here")


# speedup vs baseline: 2.6781x; 2.6781x over previous
import functools

import jax
import jax.numpy as jnp
from jax import lax
from jax.experimental import pallas as pl
from jax.experimental.pallas import tpu as pltpu
from jax.experimental.pallas import tpu_sc as plsc

N = 10000
E = 320000
C = 128

NC = 2
NS = 16
NW = NC * NS
NN = (N + NW - 1) // NW
NPAD = NN * NW
CH = 2000
NGROUP = CH // 16
NCHUNK = E // CH
CAP = 16384
BS = 256
NEG = -3.0e38


def _precompute_body(x_ref, wt_ref, wp_ref, a_ref, b_ref):
    x = x_ref[...]
    wt = wt_ref[...]
    wp = wp_ref[...]
    a_ref[...] = jax.lax.dot_general(
        x, wt, (((1,), (1,)), ((), ())), preferred_element_type=jnp.float32)
    b_ref[...] = jax.lax.dot_general(
        x, wp - wt, (((1,), (1,)), ((), ())), preferred_element_type=jnp.float32)


@jax.jit
def _precompute(x, wt, wp):
    return pl.pallas_call(
        _precompute_body,
        out_shape=[
            jax.ShapeDtypeStruct((N, C), jnp.float32),
            jax.ShapeDtypeStruct((N, C), jnp.float32),
        ],
    )(x, wt, wp)


def _scatter_max_body(a_hbm, src_hbm, dst_hbm, m_hbm,
                      m_v, dlist, rlist, srcv, dstv, gbuf, sem):
    wid = lax.axis_index("s") * NC + lax.axis_index("c")
    base = wid * NN

    def init_m(i, _):
        m_v[pl.ds(i * 16, 16)] = jnp.full((16,), NEG, jnp.float32)
        return 0
    lax.fori_loop(0, NN * C // 16, init_m, 0)

    def init_d(i, _):
        dlist[pl.ds(i * 16, 16)] = jnp.zeros((16,), jnp.int32)
        return 0
    lax.fori_loop(0, CAP // 16, init_d, 0)

    def chunk_body(ci, ptr):
        off = pl.multiple_of(ci * CH, 8)
        pltpu.sync_copy(src_hbm.at[pl.ds(off, CH)], srcv)
        pltpu.sync_copy(dst_hbm.at[pl.ds(off, CH)], dstv)

        def group_body(g, ptr):
            s = srcv[pl.ds(g * 16, 16)]
            d = dstv[pl.ds(g * 16, 16)]
            m = (s >= base) & (s < base + NN)
            pos = plsc.cumsum(jnp.where(m, 1, 0).astype(jnp.int32))
            idx = ptr + pos - 1
            plsc.store_scatter(rlist, [idx], s - base, mask=m)
            plsc.store_scatter(dlist, [idx], d, mask=m)
            return ptr + jnp.max(pos)

        return lax.fori_loop(0, NGROUP, group_body, ptr)

    count = lax.fori_loop(0, NCHUNK, chunk_body, jnp.int32(0))

    def batch_body(b, _):
        boff = pl.multiple_of(b * BS, 8)
        pltpu.async_copy(a_hbm.at[dlist.at[pl.ds(boff, BS)]], gbuf, sem).wait()
        jb = jnp.minimum(BS, count - b * BS)

        def edge_body(j, _):
            r = rlist[pl.ds(boff + j, 16)][0]
            roff = r * C
            for k in range(C // 16):
                cur = m_v[pl.ds(roff + k * 16, 16)]
                g = gbuf[j, pl.ds(k * 16, 16)]
                m_v[pl.ds(roff + k * 16, 16)] = jnp.maximum(cur, g)
            return 0

        lax.fori_loop(0, jb, edge_body, 0)
        return 0

    nb = (count + BS - 1) // BS
    lax.fori_loop(0, nb, batch_body, 0)

    pltpu.sync_copy(m_v, m_hbm.at[pl.ds(pl.multiple_of(base * C, 8), NN * C)])


@jax.jit
def _scatter_max(a, src, dst):
    mesh = plsc.VectorSubcoreMesh(
        core_axis_name="c", subcore_axis_name="s",
        num_cores=NC, num_subcores=NS)
    return pl.kernel(
        _scatter_max_body,
        out_type=jax.ShapeDtypeStruct((NPAD * C,), jnp.float32),
        mesh=mesh,
        compiler_params=pltpu.CompilerParams(needs_layout_passes=False),
        scratch_types=[
            pltpu.VMEM((NN * C,), jnp.float32),
            pltpu.VMEM((CAP,), jnp.int32),
            pltpu.VMEM((CAP + 16,), jnp.int32),
            pltpu.VMEM((CH,), jnp.int32),
            pltpu.VMEM((CH,), jnp.int32),
            pltpu.VMEM((BS, C), jnp.float32),
            pltpu.SemaphoreType.DMA,
        ],
    )(a, src, dst)


def _epilogue_body(m_ref, b_ref, gamma_ref, beta_ref, out_ref):
    t = jnp.maximum(b_ref[...] + m_ref[...], 0.0)
    mean = jnp.mean(t, axis=0, keepdims=True)
    var = jnp.mean((t - mean) ** 2, axis=0, keepdims=True)
    out = (t - mean) * jax.lax.rsqrt(var + 1e-5) * gamma_ref[...] + beta_ref[...]
    out_ref[...] = jnp.maximum(out, 0.0)


@jax.jit
def _epilogue(m, b, gamma, beta):
    return pl.pallas_call(
        _epilogue_body,
        out_shape=jax.ShapeDtypeStruct((N, C), jnp.float32),
    )(m, b, gamma, beta)


def kernel(x, src, dst, W_theta, W_phi, gamma, beta):
    a, b = _precompute(x, W_theta, W_phi)
    m_flat = _scatter_max(a, src, dst)
    m = m_flat.reshape(NPAD, C)[:N]
    return _epilogue(m, b, gamma.reshape(1, C), beta.reshape(1, C))


# speedup vs baseline: 4.8417x; 1.8079x over previous
import jax
import jax.numpy as jnp
from jax import lax
from jax.experimental import pallas as pl
from jax.experimental.pallas import tpu as pltpu
from jax.experimental.pallas import tpu_sc as plsc

N = 10000
E = 320000
C = 128

NC = 2
NS = 16
NW = NC * NS
NN = (N + NW - 1) // NW
NPAD = NN * NW
CH = 4000
NGROUP = CH // 16
NCHUNK = E // CH
CAP = 16384
BS = 128
NPADG = 10
NEG = -3.0e38


def _precompute_body(x_ref, wt_ref, wp_ref, a_ref, b_ref):
    x = x_ref[...]
    wt = wt_ref[...]
    wp = wp_ref[...]
    a_ref[...] = jax.lax.dot_general(
        x, wt, (((1,), (1,)), ((), ())), preferred_element_type=jnp.float32)
    b_ref[...] = jax.lax.dot_general(
        x, wp - wt, (((1,), (1,)), ((), ())), preferred_element_type=jnp.float32)


@jax.jit
def _precompute(x, wt, wp):
    return pl.pallas_call(
        _precompute_body,
        out_shape=[
            jax.ShapeDtypeStruct((N, C), jnp.float32),
            jax.ShapeDtypeStruct((N, C), jnp.float32),
        ],
    )(x, wt, wp)


def _scatter_max_body(a_hbm, src_hbm, dst_hbm, m_hbm,
                      m_v, dlist, rlist, srcv0, dstv0, srcv1, dstv1,
                      gbuf0, gbuf1, fsem0, fsem1, gsem0, gsem1):
    wid = lax.axis_index("s") * NC + lax.axis_index("c")
    base = wid * NN

    def init_m(i, _):
        m_v[pl.ds(i * 16, 16)] = jnp.full((16,), NEG, jnp.float32)
        return 0
    lax.fori_loop(0, NN * C // 16, init_m, 0)

    def chunk_copies(ci, sv, dv, sem):
        off = pl.multiple_of(ci * CH, 8)
        return (pltpu.make_async_copy(src_hbm.at[pl.ds(off, CH)], sv, sem),
                pltpu.make_async_copy(dst_hbm.at[pl.ds(off, CH)], dv, sem))

    def start_chunk(ci, sv, dv, sem):
        c1, c2 = chunk_copies(ci, sv, dv, sem)
        c1.start()
        c2.start()

    def wait_chunk(ci, sv, dv, sem):
        c1, c2 = chunk_copies(ci, sv, dv, sem)
        c1.wait()
        c2.wait()

    def filter_chunk(sv, dv, ptr):
        def group_body(g, ptr):
            s = sv[pl.ds(g * 16, 16)]
            d = dv[pl.ds(g * 16, 16)]
            m = (s >= base) & (s < base + NN)
            plsc.store_compressed(rlist.at[pl.ds(ptr, 16)], s - base, mask=m)
            plsc.store_compressed(dlist.at[pl.ds(ptr, 16)], d, mask=m)
            return ptr + plsc.all_reduce_population_count(m)[0]
        return lax.fori_loop(0, NGROUP, group_body, ptr)

    start_chunk(0, srcv0, dstv0, fsem0)

    def chunk_pair(t, ptr):
        c0 = 2 * t
        wait_chunk(c0, srcv0, dstv0, fsem0)
        start_chunk(c0 + 1, srcv1, dstv1, fsem1)
        ptr = filter_chunk(srcv0, dstv0, ptr)
        wait_chunk(c0 + 1, srcv1, dstv1, fsem1)

        @pl.when(c0 + 2 < NCHUNK)
        def _():
            start_chunk(c0 + 2, srcv0, dstv0, fsem0)
        return filter_chunk(srcv1, dstv1, ptr)

    count = lax.fori_loop(0, NCHUNK // 2, chunk_pair, jnp.int32(0))

    def pad_body(i, _):
        dlist[pl.ds(count + i * 16, 16)] = jnp.zeros((16,), jnp.int32)
        rlist[pl.ds(count + i * 16, 16)] = jnp.full((16,), NN, jnp.int32)
        return 0
    lax.fori_loop(0, NPADG, pad_body, 0)

    def gather_copy(b, gbuf, sem):
        boff = pl.multiple_of(b * BS, 8)
        return pltpu.make_async_copy(a_hbm.at[dlist.at[pl.ds(boff, BS)]],
                                     gbuf, sem)

    def process_batch(b, gbuf):
        boff = pl.multiple_of(b * BS, 8)

        def group_body(g, _):
            rvec = rlist[pl.ds(boff + g * 16, 16)]
            for j in range(16):
                roff = rvec[j] * C
                for k in range(C // 16):
                    cur = m_v[pl.ds(roff + k * 16, 16)]
                    gk = gbuf[g * 16 + j, pl.ds(k * 16, 16)]
                    m_v[pl.ds(roff + k * 16, 16)] = jnp.maximum(cur, gk)
            return 0
        lax.fori_loop(0, BS // 16, group_body, 0)

    nb = (count + BS - 1) // BS

    @pl.when(nb > 0)
    def _():
        gather_copy(0, gbuf0, gsem0).start()

    def batch_pair(t, _):
        b0 = 2 * t
        b1 = b0 + 1

        @pl.when(b0 < nb)
        def _():
            gather_copy(b0, gbuf0, gsem0).wait()

            @pl.when(b1 < nb)
            def _():
                gather_copy(b1, gbuf1, gsem1).start()
            process_batch(b0, gbuf0)

        @pl.when(b1 < nb)
        def _():
            gather_copy(b1, gbuf1, gsem1).wait()

            @pl.when(b1 + 1 < nb)
            def _():
                gather_copy(b1 + 1, gbuf0, gsem0).start()
            process_batch(b1, gbuf1)
        return 0

    lax.fori_loop(0, (nb + 1) // 2, batch_pair, 0)

    pltpu.sync_copy(m_v.at[pl.ds(0, NN * C)],
                    m_hbm.at[pl.ds(pl.multiple_of(base * C, 8), NN * C)])


@jax.jit
def _scatter_max(a, src, dst):
    mesh = plsc.VectorSubcoreMesh(
        core_axis_name="c", subcore_axis_name="s",
        num_cores=NC, num_subcores=NS)
    return pl.kernel(
        _scatter_max_body,
        out_type=jax.ShapeDtypeStruct((NPAD * C,), jnp.float32),
        mesh=mesh,
        compiler_params=pltpu.CompilerParams(needs_layout_passes=False),
        scratch_types=[
            pltpu.VMEM(((NN + 1) * C,), jnp.float32),
            pltpu.VMEM((CAP + 16 * NPADG + BS,), jnp.int32),
            pltpu.VMEM((CAP + 16 * NPADG + BS,), jnp.int32),
            pltpu.VMEM((CH,), jnp.int32),
            pltpu.VMEM((CH,), jnp.int32),
            pltpu.VMEM((CH,), jnp.int32),
            pltpu.VMEM((CH,), jnp.int32),
            pltpu.VMEM((BS, C), jnp.float32),
            pltpu.VMEM((BS, C), jnp.float32),
            pltpu.SemaphoreType.DMA,
            pltpu.SemaphoreType.DMA,
            pltpu.SemaphoreType.DMA,
            pltpu.SemaphoreType.DMA,
        ],
    )(a, src, dst)


def _epilogue_body(m_ref, b_ref, gamma_ref, beta_ref, out_ref):
    t = jnp.maximum(b_ref[...] + m_ref[...], 0.0)
    mean = jnp.mean(t, axis=0, keepdims=True)
    var = jnp.mean((t - mean) ** 2, axis=0, keepdims=True)
    out = (t - mean) * jax.lax.rsqrt(var + 1e-5) * gamma_ref[...] + beta_ref[...]
    out_ref[...] = jnp.maximum(out, 0.0)


@jax.jit
def _epilogue(m, b, gamma, beta):
    return pl.pallas_call(
        _epilogue_body,
        out_shape=jax.ShapeDtypeStruct((N, C), jnp.float32),
    )(m, b, gamma, beta)


def kernel(x, src, dst, W_theta, W_phi, gamma, beta):
    a, b = _precompute(x, W_theta, W_phi)
    m_flat = _scatter_max(a, src, dst)
    m = m_flat.reshape(NPAD, C)[:N]
    return _epilogue(m, b, gamma.reshape(1, C), beta.reshape(1, C))


# speedup vs baseline: 8.6237x; 1.7811x over previous
import jax
import jax.numpy as jnp
from jax import lax
from jax.experimental import pallas as pl
from jax.experimental.pallas import tpu as pltpu
from jax.experimental.pallas import tpu_sc as plsc

N = 10000
E = 320000
C = 128

NC = 2
NS = 16
NW = NC * NS
NN = (N + NW - 1) // NW
NPAD = NN * NW
CH = 3200
UNROLL = 4
NGROUP = CH // 16
NCHUNK = E // CH
CAP = 16384
BS = 128
NPADG = 10
NEG = -3.0e38


def _precompute_body(x_ref, wt_ref, wp_ref, a_ref, b_ref):
    x = x_ref[...]
    wt = wt_ref[...]
    wp = wp_ref[...]
    a_ref[...] = jax.lax.dot_general(
        x, wt, (((1,), (1,)), ((), ())), preferred_element_type=jnp.float32)
    b_ref[...] = jax.lax.dot_general(
        x, wp - wt, (((1,), (1,)), ((), ())), preferred_element_type=jnp.float32)


@jax.jit
def _precompute(x, wt, wp):
    return pl.pallas_call(
        _precompute_body,
        out_shape=[
            jax.ShapeDtypeStruct((N, C), jnp.float32),
            jax.ShapeDtypeStruct((N, C), jnp.float32),
        ],
    )(x, wt, wp)


def _scatter_max_body(a_hbm, src_hbm, dst_hbm, m_hbm,
                      m_v, dlist, rlist, srcv0, dstv0, srcv1, dstv1,
                      gbuf0, gbuf1, fsem0, fsem1, gsem0, gsem1):
    wid = lax.axis_index("s") * NC + lax.axis_index("c")
    base = wid * NN

    def init_m(i, _):
        m_v[pl.ds(i * 16, 16)] = jnp.full((16,), NEG, jnp.float32)
        return 0
    lax.fori_loop(0, NN * C // 16, init_m, 0)

    def chunk_copies(ci, sv, dv, sem):
        off = pl.multiple_of(ci * CH, 8)
        return (pltpu.make_async_copy(src_hbm.at[pl.ds(off, CH)], sv, sem),
                pltpu.make_async_copy(dst_hbm.at[pl.ds(off, CH)], dv, sem))

    def start_chunk(ci, sv, dv, sem):
        c1, c2 = chunk_copies(ci, sv, dv, sem)
        c1.start()
        c2.start()

    def wait_chunk(ci, sv, dv, sem):
        c1, c2 = chunk_copies(ci, sv, dv, sem)
        c1.wait()
        c2.wait()

    def filter_chunk(sv, dv, ptr):
        def group_body(g, ptr):
            svs, ms, cnts = [], [], []
            for u in range(UNROLL):
                s = sv[pl.ds((g * UNROLL + u) * 16, 16)]
                m = (s >= base) & (s < base + NN)
                svs.append(s)
                ms.append(m)
                cnts.append(plsc.all_reduce_population_count(m)[0])
            for u in range(UNROLL):
                d = dv[pl.ds((g * UNROLL + u) * 16, 16)]
                plsc.store_compressed(rlist.at[pl.ds(ptr, 16)],
                                      svs[u] - base, mask=ms[u])
                plsc.store_compressed(dlist.at[pl.ds(ptr, 16)], d, mask=ms[u])
                ptr = ptr + cnts[u]
            return ptr
        return lax.fori_loop(0, NGROUP // UNROLL, group_body, ptr)

    start_chunk(0, srcv0, dstv0, fsem0)

    def chunk_pair(t, ptr):
        c0 = 2 * t
        wait_chunk(c0, srcv0, dstv0, fsem0)
        start_chunk(c0 + 1, srcv1, dstv1, fsem1)
        ptr = filter_chunk(srcv0, dstv0, ptr)
        wait_chunk(c0 + 1, srcv1, dstv1, fsem1)

        @pl.when(c0 + 2 < NCHUNK)
        def _():
            start_chunk(c0 + 2, srcv0, dstv0, fsem0)
        return filter_chunk(srcv1, dstv1, ptr)

    count = lax.fori_loop(0, NCHUNK // 2, chunk_pair, jnp.int32(0))

    def pad_body(i, _):
        dlist[pl.ds(count + i * 16, 16)] = jnp.zeros((16,), jnp.int32)
        rlist[pl.ds(count + i * 16, 16)] = jnp.full((16,), NN, jnp.int32)
        return 0
    lax.fori_loop(0, NPADG, pad_body, 0)

    def gather_copy(b, gbuf, sem):
        boff = pl.multiple_of(b * BS, 8)
        return pltpu.make_async_copy(a_hbm.at[dlist.at[pl.ds(boff, BS)]],
                                     gbuf, sem)

    def process_batch(b, gbuf):
        boff = pl.multiple_of(b * BS, 8)

        def group_body(g, _):
            rvec = rlist[pl.ds(boff + g * 16, 16)]
            for j in range(16):
                roff = rvec[j] * C
                gks = [gbuf[g * 16 + j, pl.ds(k * 16, 16)]
                       for k in range(C // 16)]
                curs = [m_v[pl.ds(roff + k * 16, 16)] for k in range(C // 16)]
                for k in range(C // 16):
                    m_v[pl.ds(roff + k * 16, 16)] = jnp.maximum(curs[k], gks[k])
            return 0
        lax.fori_loop(0, BS // 16, group_body, 0)

    nb = (count + BS - 1) // BS

    @pl.when(nb > 0)
    def _():
        gather_copy(0, gbuf0, gsem0).start()

    def batch_pair(t, _):
        b0 = 2 * t
        b1 = b0 + 1

        @pl.when(b0 < nb)
        def _():
            gather_copy(b0, gbuf0, gsem0).wait()

            @pl.when(b1 < nb)
            def _():
                gather_copy(b1, gbuf1, gsem1).start()
            process_batch(b0, gbuf0)

        @pl.when(b1 < nb)
        def _():
            gather_copy(b1, gbuf1, gsem1).wait()

            @pl.when(b1 + 1 < nb)
            def _():
                gather_copy(b1 + 1, gbuf0, gsem0).start()
            process_batch(b1, gbuf1)
        return 0

    lax.fori_loop(0, (nb + 1) // 2, batch_pair, 0)

    pltpu.sync_copy(m_v.at[pl.ds(0, NN * C)],
                    m_hbm.at[pl.ds(pl.multiple_of(base * C, 8), NN * C)])


@jax.jit
def _scatter_max(a, src, dst):
    mesh = plsc.VectorSubcoreMesh(
        core_axis_name="c", subcore_axis_name="s",
        num_cores=NC, num_subcores=NS)
    return pl.kernel(
        _scatter_max_body,
        out_type=jax.ShapeDtypeStruct((NPAD * C,), jnp.float32),
        mesh=mesh,
        compiler_params=pltpu.CompilerParams(needs_layout_passes=False),
        scratch_types=[
            pltpu.VMEM(((NN + 1) * C,), jnp.float32),
            pltpu.VMEM((CAP + 16 * NPADG + BS,), jnp.int32),
            pltpu.VMEM((CAP + 16 * NPADG + BS,), jnp.int32),
            pltpu.VMEM((CH,), jnp.int32),
            pltpu.VMEM((CH,), jnp.int32),
            pltpu.VMEM((CH,), jnp.int32),
            pltpu.VMEM((CH,), jnp.int32),
            pltpu.VMEM((BS, C), jnp.float32),
            pltpu.VMEM((BS, C), jnp.float32),
            pltpu.SemaphoreType.DMA,
            pltpu.SemaphoreType.DMA,
            pltpu.SemaphoreType.DMA,
            pltpu.SemaphoreType.DMA,
        ],
    )(a, src, dst)


def _epilogue_body(m_ref, b_ref, gamma_ref, beta_ref, out_ref):
    t = jnp.maximum(b_ref[...] + m_ref[...], 0.0)
    mean = jnp.mean(t, axis=0, keepdims=True)
    var = jnp.mean((t - mean) ** 2, axis=0, keepdims=True)
    out = (t - mean) * jax.lax.rsqrt(var + 1e-5) * gamma_ref[...] + beta_ref[...]
    out_ref[...] = jnp.maximum(out, 0.0)


@jax.jit
def _epilogue(m, b, gamma, beta):
    return pl.pallas_call(
        _epilogue_body,
        out_shape=jax.ShapeDtypeStruct((N, C), jnp.float32),
    )(m, b, gamma, beta)


def kernel(x, src, dst, W_theta, W_phi, gamma, beta):
    a, b = _precompute(x, W_theta, W_phi)
    m_flat = _scatter_max(a, src, dst)
    m = m_flat.reshape(NPAD, C)[:N]
    return _epilogue(m, b, gamma.reshape(1, C), beta.reshape(1, C))


# speedup vs baseline: 9.5983x; 1.1130x over previous
import jax
import jax.numpy as jnp
from jax import lax
from jax.experimental import pallas as pl
from jax.experimental.pallas import tpu as pltpu
from jax.experimental.pallas import tpu_sc as plsc

N = 10000
E = 320000
C = 128

NC = 2
NS = 16
NW = NC * NS
NN = (N + NW - 1) // NW
NPAD = NN * NW
CH = 3200
UNROLL = 8
NGROUP = CH // 16
NCHUNK = E // CH
CAP = 16384
BS = 128
NPADG = 10
NEG = -3.0e38


def _precompute_body(x_ref, wt_ref, wp_ref, a_ref, b_ref):
    x = x_ref[...]
    wt = wt_ref[...]
    wp = wp_ref[...]
    a_ref[...] = jax.lax.dot_general(
        x, wt, (((1,), (1,)), ((), ())), preferred_element_type=jnp.float32)
    b_ref[...] = jax.lax.dot_general(
        x, wp - wt, (((1,), (1,)), ((), ())), preferred_element_type=jnp.float32)


@jax.jit
def _precompute(x, wt, wp):
    return pl.pallas_call(
        _precompute_body,
        out_shape=[
            jax.ShapeDtypeStruct((N, C), jnp.float32),
            jax.ShapeDtypeStruct((N, C), jnp.float32),
        ],
    )(x, wt, wp)


def _scatter_max_body(a_hbm, pk_hbm, m_hbm,
                      m_v, dlist, rlist, plist, pkv0, pkv1,
                      gbuf0, gbuf1, fsem0, fsem1, gsem0, gsem1):
    wid = lax.axis_index("s") * NC + lax.axis_index("c")
    base = wid * NN

    def init_m(i, _):
        m_v[pl.ds(i * 16, 16)] = jnp.full((16,), NEG, jnp.float32)
        return 0
    lax.fori_loop(0, (NN + 1) * C // 16, init_m, 0)

    def chunk_copy(ci, pv, sem):
        off = pl.multiple_of(ci * CH, 8)
        return pltpu.make_async_copy(pk_hbm.at[pl.ds(off, CH)], pv, sem)

    def filter_chunk(pv, ptr):
        def group_body(g, ptr):
            pws, ms, cnts = [], [], []
            for u in range(UNROLL):
                pw = pv[pl.ds((g * UNROLL + u) * 16, 16)]
                sl = pw & jnp.int32(0xFFFF)
                m = (sl >= base) & (sl < base + NN)
                pws.append(pw)
                ms.append(m)
                cnts.append(plsc.all_reduce_population_count(m)[0])
            for u in range(UNROLL):
                plsc.store_compressed(plist.at[pl.ds(ptr, 16)], pws[u],
                                      mask=ms[u])
                ptr = ptr + cnts[u]
            return ptr
        return lax.fori_loop(0, NGROUP // UNROLL, group_body, ptr)

    chunk_copy(0, pkv0, fsem0).start()

    def chunk_pair(t, ptr):
        c0 = 2 * t
        chunk_copy(c0, pkv0, fsem0).wait()
        chunk_copy(c0 + 1, pkv1, fsem1).start()
        ptr = filter_chunk(pkv0, ptr)
        chunk_copy(c0 + 1, pkv1, fsem1).wait()

        @pl.when(c0 + 2 < NCHUNK)
        def _():
            chunk_copy(c0 + 2, pkv0, fsem0).start()
        return filter_chunk(pkv1, ptr)

    count = lax.fori_loop(0, NCHUNK // 2, chunk_pair, jnp.int32(0))

    def pad_body(i, _):
        plist[pl.ds(count + i * 16, 16)] = jnp.full(
            (16,), (NN + base) | 0, jnp.int32)
        return 0
    lax.fori_loop(0, NPADG, pad_body, 0)

    def split_body(i, _):
        pw = plist[pl.ds(i * 16, 16)]
        rlist[pl.ds(i * 16, 16)] = (pw & jnp.int32(0xFFFF)) - base
        dlist[pl.ds(i * 16, 16)] = lax.shift_right_logical(pw, 16)
        return 0
    lax.fori_loop(0, (count + NPADG * 16 + 15) // 16, split_body, 0)

    def gather_copy(b, gbuf, sem):
        boff = pl.multiple_of(b * BS, 8)
        return pltpu.make_async_copy(a_hbm.at[dlist.at[pl.ds(boff, BS)]],
                                     gbuf, sem)

    def process_batch(b, gbuf):
        boff = pl.multiple_of(b * BS, 8)

        def group_body(g, _):
            rvec = rlist[pl.ds(boff + g * 16, 16)]
            for j in range(16):
                roff = rvec[j] * C
                gks = [gbuf[g * 16 + j, pl.ds(k * 16, 16)]
                       for k in range(C // 16)]
                curs = [m_v[pl.ds(roff + k * 16, 16)] for k in range(C // 16)]
                for k in range(C // 16):
                    m_v[pl.ds(roff + k * 16, 16)] = jnp.maximum(curs[k], gks[k])
            return 0
        lax.fori_loop(0, BS // 16, group_body, 0)

    nb = (count + BS - 1) // BS

    @pl.when(nb > 0)
    def _():
        gather_copy(0, gbuf0, gsem0).start()

    def batch_pair(t, _):
        b0 = 2 * t
        b1 = b0 + 1

        @pl.when(b0 < nb)
        def _():
            gather_copy(b0, gbuf0, gsem0).wait()

            @pl.when(b1 < nb)
            def _():
                gather_copy(b1, gbuf1, gsem1).start()
            process_batch(b0, gbuf0)

        @pl.when(b1 < nb)
        def _():
            gather_copy(b1, gbuf1, gsem1).wait()

            @pl.when(b1 + 1 < nb)
            def _():
                gather_copy(b1 + 1, gbuf0, gsem0).start()
            process_batch(b1, gbuf1)
        return 0

    lax.fori_loop(0, (nb + 1) // 2, batch_pair, 0)

    pltpu.sync_copy(m_v.at[pl.ds(0, NN * C)],
                    m_hbm.at[pl.ds(pl.multiple_of(base * C, 8), NN * C)])


@jax.jit
def _scatter_max(a, pk):
    mesh = plsc.VectorSubcoreMesh(
        core_axis_name="c", subcore_axis_name="s",
        num_cores=NC, num_subcores=NS)
    return pl.kernel(
        _scatter_max_body,
        out_type=jax.ShapeDtypeStruct((NPAD * C,), jnp.float32),
        mesh=mesh,
        compiler_params=pltpu.CompilerParams(needs_layout_passes=False),
        scratch_types=[
            pltpu.VMEM(((NN + 1) * C,), jnp.float32),
            pltpu.VMEM((CAP + 16 * NPADG + BS,), jnp.int32),
            pltpu.VMEM((CAP + 16 * NPADG + BS,), jnp.int32),
            pltpu.VMEM((CAP + 16 * NPADG + BS,), jnp.int32),
            pltpu.VMEM((CH,), jnp.int32),
            pltpu.VMEM((CH,), jnp.int32),
            pltpu.VMEM((BS, C), jnp.float32),
            pltpu.VMEM((BS, C), jnp.float32),
            pltpu.SemaphoreType.DMA,
            pltpu.SemaphoreType.DMA,
            pltpu.SemaphoreType.DMA,
            pltpu.SemaphoreType.DMA,
        ],
    )(a, pk)


def _epilogue_body(m_ref, b_ref, gamma_ref, beta_ref, out_ref):
    t = jnp.maximum(b_ref[...] + m_ref[...], 0.0)
    mean = jnp.mean(t, axis=0, keepdims=True)
    var = jnp.mean((t - mean) ** 2, axis=0, keepdims=True)
    out = (t - mean) * jax.lax.rsqrt(var + 1e-5) * gamma_ref[...] + beta_ref[...]
    out_ref[...] = jnp.maximum(out, 0.0)


@jax.jit
def _epilogue(m, b, gamma, beta):
    return pl.pallas_call(
        _epilogue_body,
        out_shape=jax.ShapeDtypeStruct((N, C), jnp.float32),
    )(m, b, gamma, beta)


def kernel(x, src, dst, W_theta, W_phi, gamma, beta):
    a, b = _precompute(x, W_theta, W_phi)
    pk = src | (dst << 16)
    m_flat = _scatter_max(a, pk)
    m = m_flat.reshape(NPAD, C)[:N]
    return _epilogue(m, b, gamma.reshape(1, C), beta.reshape(1, C))
